# baseline (speedup 1.0000x reference)
import jax
import jax.numpy as jnp
from jax.experimental import pallas as pl
from jax.experimental.pallas import tpu as pltpu

_BN = 128


def _assign_kernel(x_ref, c_ref, o_ref, c2_ref):
    i = pl.program_id(0)

    @pl.when(i == 0)
    def _():
        cc = c_ref[...]
        c2_ref[...] = jnp.sum(cc * cc, axis=1)[None, :]

    x = x_ref[...]
    dot = jax.lax.dot_general(
        x, c_ref[...], (((1,), (1,)), ((), ())),
        preferred_element_type=jnp.float32)
    f2 = jnp.sum(x * x, axis=1, keepdims=True)
    d2 = f2 + c2_ref[...] - 2.0 * dot
    dist = jnp.sqrt(jnp.maximum(d2, 0.0))
    m = jnp.min(dist, axis=1, keepdims=True)
    ids = jax.lax.broadcasted_iota(jnp.int32, dist.shape, 1)
    big = jnp.int32(dist.shape[1])
    idx = jnp.min(jnp.where(dist <= m, ids, big), axis=1)
    o_ref[0, 0, :] = idx.astype(jnp.int32)


def kernel(inp, centroids):
    b, t, c = inp.shape
    k = centroids.shape[0]
    n = b * t
    nb = n // _BN
    x = inp.reshape(n, c)
    out = pl.pallas_call(
        _assign_kernel,
        grid=(nb,),
        in_specs=[
            pl.BlockSpec((_BN, c), lambda i: (i, 0)),
            pl.BlockSpec((k, c), lambda i: (0, 0)),
        ],
        out_specs=pl.BlockSpec((1, 1, _BN), lambda i: (i, 0, 0)),
        out_shape=jax.ShapeDtypeStruct((nb, 1, _BN), jnp.int32),
        scratch_shapes=[pltpu.VMEM((1, k), jnp.float32)],
    )(x, centroids)
    return out.reshape(b, t)


# speedup vs baseline: 1.3433x; 1.3433x over previous
import jax
import jax.numpy as jnp
from jax.experimental import pallas as pl
from jax.experimental.pallas import tpu as pltpu

_BN = 128


def _assign_kernel(x_ref, c_ref, o_ref, c2_ref):
    i = pl.program_id(0)

    @pl.when(i == 0)
    def _():
        cc = c_ref[...]
        c2_ref[...] = jnp.sum(cc * cc, axis=1)[None, :]

    xm2 = x_ref[...] * -2.0
    dot = jax.lax.dot_general(
        xm2, c_ref[...], (((1,), (1,)), ((), ())),
        preferred_element_type=jnp.float32)
    score = dot + c2_ref[...]
    m = jnp.min(score, axis=1, keepdims=True)
    ids = jax.lax.broadcasted_iota(jnp.int32, score.shape, 1)
    big = jnp.int32(score.shape[1])
    idx = jnp.min(jnp.where(score <= m, ids, big), axis=1)
    o_ref[0, 0, :] = idx.astype(jnp.int32)


def kernel(inp, centroids):
    b, t, c = inp.shape
    k = centroids.shape[0]
    n = b * t
    nb = n // _BN
    x = inp.reshape(n, c)
    out = pl.pallas_call(
        _assign_kernel,
        grid=(nb,),
        in_specs=[
            pl.BlockSpec((_BN, c), lambda i: (i, 0)),
            pl.BlockSpec((k, c), lambda i: (0, 0)),
        ],
        out_specs=pl.BlockSpec((1, 1, _BN), lambda i: (i, 0, 0)),
        out_shape=jax.ShapeDtypeStruct((nb, 1, _BN), jnp.int32),
        scratch_shapes=[pltpu.VMEM((1, k), jnp.float32)],
    )(x, centroids)
    return out.reshape(b, t)


# speedup vs baseline: 1.6021x; 1.1927x over previous
import jax
import jax.numpy as jnp
from jax.experimental import pallas as pl
from jax.experimental.pallas import tpu as pltpu

_BN = 128
_BK = 2048


def _assign_kernel(x_ref, c_ref, o_ref, c2_ref):
    i = pl.program_id(0)

    @pl.when(i == 0)
    def _():
        cc = c_ref[...]
        c2_ref[...] = jnp.sum(cc * cc, axis=1)[None, :]

    xm2 = x_ref[...] * -2.0
    k = c_ref.shape[0]
    bn = x_ref.shape[0]
    ids = jax.lax.broadcasted_iota(jnp.int32, (bn, _BK), 1).astype(jnp.float32)
    big = jnp.float32(3e9)
    best_m = None
    best_i = None
    for j in range(0, k, _BK):
        dot = jax.lax.dot_general(
            xm2, c_ref[j:j + _BK, :], (((1,), (1,)), ((), ())),
            preferred_element_type=jnp.float32)
        score = dot + c2_ref[:, j:j + _BK]
        m = jnp.min(score, axis=1)
        idx = jnp.min(
            jnp.where(score <= m[:, None], ids, big), axis=1) + jnp.float32(j)
        if best_m is None:
            best_m, best_i = m, idx
        else:
            take = m < best_m
            best_m = jnp.where(take, m, best_m)
            best_i = jnp.where(take, idx, best_i)
    o_ref[0, 0, :] = best_i.astype(jnp.int32)


def kernel(inp, centroids):
    b, t, c = inp.shape
    k = centroids.shape[0]
    n = b * t
    nb = n // _BN
    x = inp.reshape(n, c)
    out = pl.pallas_call(
        _assign_kernel,
        grid=(nb,),
        in_specs=[
            pl.BlockSpec((_BN, c), lambda i: (i, 0)),
            pl.BlockSpec((k, c), lambda i: (0, 0)),
        ],
        out_specs=pl.BlockSpec((1, 1, _BN), lambda i: (i, 0, 0)),
        out_shape=jax.ShapeDtypeStruct((nb, 1, _BN), jnp.int32),
        scratch_shapes=[pltpu.VMEM((1, k), jnp.float32)],
    )(x, centroids)
    return out.reshape(b, t)


# speedup vs baseline: 2.4232x; 1.5125x over previous
import jax
import jax.numpy as jnp
from jax.experimental import pallas as pl
from jax.experimental.pallas import tpu as pltpu

_BN = 256
_BK = 2048


def _assign_kernel(x_ref, c_ref, o_ref, c2_ref):
    i = pl.program_id(0)

    @pl.when(i == 0)
    def _():
        cc = c_ref[...]
        ones = jnp.ones((1, cc.shape[1]), jnp.float32)
        c2_ref[...] = jax.lax.dot_general(
            ones, cc * cc, (((1,), (1,)), ((), ())),
            preferred_element_type=jnp.float32)

    xm2 = x_ref[...] * -2.0
    k = c_ref.shape[0]
    bn = x_ref.shape[0]
    ids = jax.lax.broadcasted_iota(jnp.int32, (bn, _BK), 1).astype(jnp.float32)
    big = jnp.float32(3e9)
    best_m = None
    best_i = None
    for j in range(0, k, _BK):
        dot = jax.lax.dot_general(
            xm2, c_ref[j:j + _BK, :], (((1,), (1,)), ((), ())),
            preferred_element_type=jnp.float32)
        score = dot + c2_ref[:, j:j + _BK]
        m = jnp.min(score, axis=1)
        idx = jnp.min(
            jnp.where(score <= m[:, None], ids, big), axis=1) + jnp.float32(j)
        if best_m is None:
            best_m, best_i = m, idx
        else:
            take = m < best_m
            best_m = jnp.where(take, m, best_m)
            best_i = jnp.where(take, idx, best_i)
    o_ref[0, 0, :] = best_i.astype(jnp.int32)


def kernel(inp, centroids):
    b, t, c = inp.shape
    k = centroids.shape[0]
    n = b * t
    nb = n // _BN
    x = inp.reshape(n, c)
    out = pl.pallas_call(
        _assign_kernel,
        grid=(nb,),
        in_specs=[
            pl.BlockSpec((_BN, c), lambda i: (i, 0)),
            pl.BlockSpec((k, c), lambda i: (0, 0)),
        ],
        out_specs=pl.BlockSpec((1, 1, _BN), lambda i: (i, 0, 0)),
        out_shape=jax.ShapeDtypeStruct((nb, 1, _BN), jnp.int32),
        scratch_shapes=[pltpu.VMEM((1, k), jnp.float32)],
    )(x, centroids)
    return out.reshape(b, t)
